# Initial kernel scaffold; baseline (speedup 1.0000x reference)
import jax
import jax.numpy as jnp
from jax.experimental import pallas as pl


def kernel(x, edge_index, edge_attr, batch, W1, b1, p1, W2, b2, p2, Wl1, bl1, Wl2, bl2, Wl3, bl3):
    raise NotImplementedError("write your pallas kernel here")


# speedup vs baseline: 1.0000x; 1.0000x over previous
import jax
import jax.numpy as jnp
from jax.experimental import pallas as pl
from jax.experimental.pallas import tpu as pltpu

_NUM_GRAPHS = 128
_RATIO = 0.5


def _pool_plan(batch, valid, ratio, num_graphs):
    ones = jnp.ones_like(batch)
    full = jax.ops.segment_sum(ones, batch, num_segments=num_graphs)
    starts = jnp.cumsum(full) - full
    vcounts = jax.ops.segment_sum(jnp.where(valid, ones, jnp.zeros_like(ones)), batch, num_segments=num_graphs)
    k = jnp.ceil(ratio * vcounts).astype(batch.dtype)
    rank = jnp.arange(batch.shape[0], dtype=batch.dtype) - starts[batch]
    keep = rank < k[batch]
    return keep, k


def _gcn_conv(x, src, dst, ew, W, b):
    n = x.shape[0]
    h = x @ W
    li = jnp.arange(n, dtype=src.dtype)
    s = jnp.concatenate([src, li])
    d = jnp.concatenate([dst, li])
    w = jnp.concatenate([ew, jnp.ones((n,), x.dtype)])
    deg = jax.ops.segment_sum(w, d, num_segments=n)
    dinv = jax.lax.rsqrt(jnp.maximum(deg, 1e-12))
    norm = dinv[s] * w * dinv[d]
    out = jax.ops.segment_sum(h[s] * norm[:, None], d, num_segments=n)
    return out + b


def _topk_pool(x, src, dst, ew, batch, p, valid, keep):
    score = jnp.tanh((x @ p) / jnp.linalg.norm(p))
    skey = jnp.where(valid, score, -jnp.inf)
    perm = jnp.lexsort((-skey, batch))
    new_x = x[perm] * score[perm][:, None]
    n = x.shape[0]
    mask = jnp.zeros((n,), bool).at[perm].set(keep)
    new_idx = jnp.zeros((n,), src.dtype).at[perm].set(jnp.arange(n, dtype=src.dtype))
    evalid = mask[src] & mask[dst]
    new_src = jnp.where(evalid, new_idx[src], 0)
    new_dst = jnp.where(evalid, new_idx[dst], 0)
    new_ew = jnp.where(evalid, ew, 0.0)
    return new_x, new_src, new_dst, new_ew, keep


def _readout(x, batch, valid, k, num_graphs):
    counts = k.astype(jnp.float32)
    mx = jax.ops.segment_max(jnp.where(valid[:, None], x, -jnp.inf), batch, num_segments=num_graphs)
    mx = jnp.where(counts[:, None] > 0, mx, 0.0)
    mean = jax.ops.segment_sum(jnp.where(valid[:, None], x, 0.0), batch, num_segments=num_graphs) / jnp.maximum(counts, 1.0)[:, None]
    return jnp.concatenate([mx, mean], axis=1)


def _head_body(x1_ref, x2_ref, w1_ref, b1_ref, w2_ref, b2_ref, w3_ref, b3_ref, o_ref):
    z = jnp.maximum(x1_ref[...] + x2_ref[...], 0.0)
    z = jnp.maximum(jnp.dot(z, w1_ref[...], preferred_element_type=jnp.float32) + b1_ref[...], 0.0)
    z = jnp.maximum(jnp.dot(z, w2_ref[...], preferred_element_type=jnp.float32) + b2_ref[...], 0.0)
    z = jnp.dot(z, w3_ref[...], preferred_element_type=jnp.float32) + b3_ref[...]
    o_ref[...] = jax.nn.sigmoid(z)


def _head(x1, x2, Wl1, bl1, Wl2, bl2, Wl3, bl3):
    g = x1.shape[0]
    c = Wl3.shape[1]
    return pl.pallas_call(
        _head_body,
        out_shape=jax.ShapeDtypeStruct((g, c), jnp.float32),
    )(x1, x2, Wl1, bl1.reshape(1, -1), Wl2, bl2.reshape(1, -1), Wl3, bl3.reshape(1, -1))


def kernel(x, edge_index, edge_attr, batch, W1, b1, p1, W2, b2, p2, Wl1, bl1, Wl2, bl2, Wl3, bl3):
    src, dst = edge_index[0], edge_index[1]
    h = jax.nn.relu(_gcn_conv(x, src, dst, edge_attr, W1, b1))
    valid0 = jnp.ones((x.shape[0],), bool)
    keep1, k1 = _pool_plan(batch, valid0, _RATIO, _NUM_GRAPHS)
    h, src, dst, ew, v1 = _topk_pool(h, src, dst, edge_attr, batch, p1, valid0, keep1)
    x1 = _readout(h, batch, v1, k1, _NUM_GRAPHS)
    h = jax.nn.relu(_gcn_conv(h, src, dst, ew, W2, b2))
    keep2, k2 = _pool_plan(batch, v1, _RATIO, _NUM_GRAPHS)
    h, src, dst, ew, v2 = _topk_pool(h, src, dst, ew, batch, p2, v1, keep2)
    x2 = _readout(h, batch, v2, k2, _NUM_GRAPHS)
    return _head(x1, x2, Wl1, bl1, Wl2, bl2, Wl3, bl3)


# speedup vs baseline: 2.2190x; 2.2190x over previous
import jax
import jax.numpy as jnp
from jax.experimental import pallas as pl
from jax.experimental.pallas import tpu as pltpu

_G = 128
_RATIO = 0.5


def _topk_mask(score, batch, valid):
    n = score.shape[0]
    ones = jnp.ones_like(batch)
    full = jax.ops.segment_sum(ones, batch, num_segments=_G)
    starts = jnp.cumsum(full) - full
    vcounts = jax.ops.segment_sum(jnp.where(valid, ones, jnp.zeros_like(ones)), batch, num_segments=_G)
    k = jnp.ceil(_RATIO * vcounts).astype(batch.dtype)
    rank = jnp.arange(n, dtype=batch.dtype) - starts[batch]
    keepp = rank < k[batch]
    skey = jnp.where(valid, score, -jnp.inf)
    perm = jnp.lexsort((-skey, batch))
    mask = jnp.zeros((n,), bool).at[perm].set(keepp)
    return mask, k


def _gcn_conv(x, src, dst, ew, W, b):
    n = x.shape[0]
    h = x @ W
    deg = jax.ops.segment_sum(ew, dst, num_segments=n) + 1.0
    dinv = jax.lax.rsqrt(jnp.maximum(deg, 1e-12))
    acc = jax.ops.segment_sum(h[src] * (ew * dinv[src])[:, None], dst, num_segments=n)
    return dinv[:, None] * acc + (dinv * dinv)[:, None] * h + b


def _readout(x, batch, valid, k):
    counts = k.astype(jnp.float32)
    mx = jax.ops.segment_max(jnp.where(valid[:, None], x, -jnp.inf), batch, num_segments=_G)
    mx = jnp.where(counts[:, None] > 0, mx, 0.0)
    mean = jax.ops.segment_sum(jnp.where(valid[:, None], x, 0.0), batch, num_segments=_G) / jnp.maximum(counts, 1.0)[:, None]
    return jnp.concatenate([mx, mean], axis=1)


def _head_body(x1_ref, x2_ref, w1_ref, b1_ref, w2_ref, b2_ref, w3_ref, b3_ref, o_ref):
    z = jnp.maximum(x1_ref[...] + x2_ref[...], 0.0)
    z = jnp.maximum(jnp.dot(z, w1_ref[...], preferred_element_type=jnp.float32) + b1_ref[...], 0.0)
    z = jnp.maximum(jnp.dot(z, w2_ref[...], preferred_element_type=jnp.float32) + b2_ref[...], 0.0)
    z = jnp.dot(z, w3_ref[...], preferred_element_type=jnp.float32) + b3_ref[...]
    o_ref[...] = jax.nn.sigmoid(z)


def _head(x1, x2, Wl1, bl1, Wl2, bl2, Wl3, bl3):
    g = x1.shape[0]
    c = Wl3.shape[1]
    return pl.pallas_call(
        _head_body,
        out_shape=jax.ShapeDtypeStruct((g, c), jnp.float32),
    )(x1, x2, Wl1, bl1.reshape(1, -1), Wl2, bl2.reshape(1, -1), Wl3, bl3.reshape(1, -1))


def kernel(x, edge_index, edge_attr, batch, W1, b1, p1, W2, b2, p2, Wl1, bl1, Wl2, bl2, Wl3, bl3):
    src, dst = edge_index[0], edge_index[1]
    n = x.shape[0]
    h = jax.nn.relu(_gcn_conv(x, src, dst, edge_attr, W1, b1))
    score1 = jnp.tanh((h @ p1) / jnp.linalg.norm(p1))
    valid0 = jnp.ones((n,), bool)
    keep1, k1 = _topk_mask(score1, batch, valid0)
    h = h * score1[:, None]
    ew = jnp.where(keep1[src] & keep1[dst], edge_attr, 0.0)
    x1 = _readout(h, batch, keep1, k1)
    h = jax.nn.relu(_gcn_conv(h, src, dst, ew, W2, b2))
    score2 = jnp.tanh((h @ p2) / jnp.linalg.norm(p2))
    keep2, k2 = _topk_mask(score2, batch, keep1)
    h = h * score2[:, None]
    x2 = _readout(h, batch, keep2, k2)
    return _head(x1, x2, Wl1, bl1, Wl2, bl2, Wl3, bl3)


# speedup vs baseline: 10.4514x; 4.7099x over previous
import functools

import jax
import jax.numpy as jnp
from jax import lax
from jax.experimental import pallas as pl
from jax.experimental.pallas import tpu as pltpu
from jax.experimental.pallas import tpu_sc as plsc

_G = 128
_RATIO = 0.5
_NC = 2
_NS = 16
_NW = _NC * _NS
_K = 80
_NP = 10240


def _make_deg_kernel(e):
    epw = e // _NW
    rounds = epw // _K
    nr = _NP // 128
    mesh = plsc.VectorSubcoreMesh(core_axis_name="c", subcore_axis_name="s")

    @functools.partial(
        pl.kernel,
        mesh=mesh,
        compiler_params=pltpu.CompilerParams(needs_layout_passes=False),
        out_type=[
            jax.ShapeDtypeStruct((_NW, nr, 128), jnp.float32),
            jax.ShapeDtypeStruct((e,), jnp.float32),
        ],
        scratch_types=[
            pltpu.VMEM((_K,), jnp.int32),
            pltpu.VMEM((_K,), jnp.int32),
            pltpu.VMEM((_K,), jnp.float32),
            pltpu.VMEM((nr, 128), jnp.float32),
            pltpu.VMEM((nr, 128), jnp.float32),
        ],
    )
    def k(src_hbm, dst_hbm, ew_hbm, keep_hbm, degp_hbm, ew2_hbm,
          sidx, didx, ewv, keepv, degv):
        cid = lax.axis_index("c")
        sid = lax.axis_index("s")
        wid = cid * _NS + sid
        base = wid * epw
        iota = lax.iota(jnp.int32, 16)
        zeros = jnp.zeros((16,), jnp.float32)

        pltpu.sync_copy(keep_hbm, keepv)

        def zb(r, c):
            rsp = jnp.full((16,), r, jnp.int32)
            for f in range(8):
                plsc.store_scatter(degv, [rsp, f * 16 + iota], zeros)
            return c

        lax.fori_loop(0, nr, zb, 0)

        def rb(j, c):
            off = base + j * _K
            pltpu.sync_copy(src_hbm.at[pl.ds(off, _K)], sidx)
            pltpu.sync_copy(dst_hbm.at[pl.ds(off, _K)], didx)
            pltpu.sync_copy(ew_hbm.at[pl.ds(off, _K)], ewv)
            for g in range(_K // 16):
                s16 = sidx[pl.ds(g * 16, 16)]
                d16 = didx[pl.ds(g * 16, 16)]
                w16 = ewv[pl.ds(g * 16, 16)]
                ks = plsc.load_gather(keepv, [s16 >> 7, s16 & 127])
                kd = plsc.load_gather(keepv, [d16 >> 7, d16 & 127])
                w2 = w16 * ks * kd
                ewv[pl.ds(g * 16, 16)] = w2
                plsc.addupdate_scatter(degv, [d16 >> 7, d16 & 127], w2)
            pltpu.sync_copy(ewv, ew2_hbm.at[pl.ds(off, _K)])
            return c

        lax.fori_loop(0, rounds, rb, 0)
        pltpu.sync_copy(degv, degp_hbm.at[wid])

    return k


def _make_msg_kernel(e):
    epw = e // _NW
    rounds = epw // _K
    nps = _NP // _NS
    zc = 128
    mesh = plsc.VectorSubcoreMesh(core_axis_name="c", subcore_axis_name="s")

    @functools.partial(
        pl.kernel,
        mesh=mesh,
        compiler_params=pltpu.CompilerParams(needs_layout_passes=False),
        out_type=jax.ShapeDtypeStruct((_NC, _NP, 128), jnp.float32),
        scratch_types=[
            pltpu.VMEM((_K,), jnp.int32),
            pltpu.VMEM((_K,), jnp.int32),
            pltpu.VMEM((_K,), jnp.float32),
            pltpu.VMEM((_K, 128), jnp.float32),
            pltpu.VMEM((zc, 128), jnp.float32),
            pltpu.VMEM_SHARED((_NP, 128), jnp.float32),
            pltpu.SemaphoreType.DMA,
        ],
    )
    def k(g_hbm, src_hbm, dst_hbm, ew_hbm, out_hbm,
          sidx, didx, ewv, rows, zbuf, acc, sem):
        cid = lax.axis_index("c")
        sid = lax.axis_index("s")
        wid = cid * _NS + sid
        base = wid * epw
        iota = lax.iota(jnp.int32, 16)
        zeros = jnp.zeros((16,), jnp.float32)

        def zb(r, c):
            rsp = jnp.full((16,), r, jnp.int32)
            for f in range(8):
                plsc.store_scatter(zbuf, [rsp, f * 16 + iota], zeros)
            return c

        lax.fori_loop(0, zc, zb, 0)
        for t in range(nps // zc):
            pltpu.sync_copy(zbuf, acc.at[pl.ds(sid * nps + t * zc, zc)])
        plsc.subcore_barrier()

        def rb(j, c):
            off = base + j * _K
            pltpu.sync_copy(src_hbm.at[pl.ds(off, _K)], sidx)
            pltpu.sync_copy(dst_hbm.at[pl.ds(off, _K)], didx)
            pltpu.sync_copy(ew_hbm.at[pl.ds(off, _K)], ewv)
            pltpu.async_copy(g_hbm.at[sidx], rows, sem).wait()

            def rowb(r, c2):
                rsp = jnp.full((16,), r, jnp.int32)
                coef = plsc.load_gather(ewv, [rsp])
                for f in range(8):
                    ci = f * 16 + iota
                    v = plsc.load_gather(rows, [rsp, ci])
                    plsc.store_scatter(rows, [rsp, ci], v * coef)
                return c2

            lax.fori_loop(0, _K, rowb, 0)
            pltpu.sync_copy(rows, acc.at[didx], add=True)
            return c

        lax.fori_loop(0, rounds, rb, 0)
        plsc.subcore_barrier()
        for t in range(nps // zc):
            sl = pl.ds(sid * nps + t * zc, zc)
            pltpu.sync_copy(acc.at[sl], out_hbm.at[cid, sl])

    return k


def _mm_body(x_ref, w_ref, o_ref):
    o_ref[...] = jnp.dot(x_ref[...], w_ref[...],
                         preferred_element_type=jnp.float32)


def _mm(x, W):
    return pl.pallas_call(
        _mm_body,
        out_shape=jax.ShapeDtypeStruct((x.shape[0], W.shape[1]), jnp.float32),
    )(x, W)


def _prep_body(degpt_ref, h_ref, dinv_ref, g_ref):
    deg = jnp.sum(degpt_ref[...], axis=1, keepdims=True) + 1.0
    dinv = lax.rsqrt(jnp.maximum(deg, 1e-12))
    dinv_ref[...] = dinv
    g_ref[...] = dinv * h_ref[...]


def _prep(degpt, h):
    n = h.shape[0]
    return pl.pallas_call(
        _prep_body,
        out_shape=(jax.ShapeDtypeStruct((n, 1), jnp.float32),
                   jax.ShapeDtypeStruct((n, 128), jnp.float32)),
    )(degpt, h)


def _epi_body(a0_ref, a1_ref, h_ref, dinv_ref, b_ref, p_ref, hp_ref, s_ref):
    dinv = dinv_ref[...]
    out = dinv * (a0_ref[...] + a1_ref[...]) \
        + (dinv * dinv) * h_ref[...] + b_ref[...]
    hf = jnp.maximum(out, 0.0)
    p = p_ref[...]
    pn = p * lax.rsqrt(jnp.sum(p * p))
    s = jnp.tanh(jnp.dot(hf, pn, preferred_element_type=jnp.float32))
    hp_ref[...] = hf * s
    s_ref[...] = s


def _epi(a0, a1, h, dinv, b, p):
    n = h.shape[0]
    return pl.pallas_call(
        _epi_body,
        out_shape=(jax.ShapeDtypeStruct((n, 128), jnp.float32),
                   jax.ShapeDtypeStruct((n, 1), jnp.float32)),
    )(a0, a1, h, dinv, b.reshape(1, -1), p.reshape(-1, 1))


def _head_body(x1_ref, x2_ref, w1_ref, b1_ref, w2_ref, b2_ref, w3_ref, b3_ref, o_ref):
    z = jnp.maximum(x1_ref[...] + x2_ref[...], 0.0)
    z = jnp.maximum(jnp.dot(z, w1_ref[...], preferred_element_type=jnp.float32) + b1_ref[...], 0.0)
    z = jnp.maximum(jnp.dot(z, w2_ref[...], preferred_element_type=jnp.float32) + b2_ref[...], 0.0)
    z = jnp.dot(z, w3_ref[...], preferred_element_type=jnp.float32) + b3_ref[...]
    o_ref[...] = jax.nn.sigmoid(z)


def _head(x1, x2, Wl1, bl1, Wl2, bl2, Wl3, bl3):
    g = x1.shape[0]
    c = Wl3.shape[1]
    return pl.pallas_call(
        _head_body,
        out_shape=jax.ShapeDtypeStruct((g, c), jnp.float32),
    )(x1, x2, Wl1, bl1.reshape(1, -1), Wl2, bl2.reshape(1, -1), Wl3, bl3.reshape(1, -1))


def _topk_mask(score, batch, valid):
    n = score.shape[0]
    ones = jnp.ones_like(batch)
    full = jax.ops.segment_sum(ones, batch, num_segments=_G)
    starts = jnp.cumsum(full) - full
    vcounts = jax.ops.segment_sum(jnp.where(valid, ones, jnp.zeros_like(ones)), batch, num_segments=_G)
    k = jnp.ceil(_RATIO * vcounts).astype(batch.dtype)
    rank = jnp.arange(n, dtype=batch.dtype) - starts[batch]
    keepp = rank < k[batch]
    skey = jnp.where(valid, score, -jnp.inf)
    perm = jnp.lexsort((-skey, batch))
    mask = jnp.zeros((n,), bool).at[perm].set(keepp)
    return mask, k


def _readout(x, batch, valid, k):
    counts = k.astype(jnp.float32)
    mx = jax.ops.segment_max(jnp.where(valid[:, None], x, -jnp.inf), batch, num_segments=_G)
    mx = jnp.where(counts[:, None] > 0, mx, 0.0)
    mean = jax.ops.segment_sum(jnp.where(valid[:, None], x, 0.0), batch, num_segments=_G) / jnp.maximum(counts, 1.0)[:, None]
    return jnp.concatenate([mx, mean], axis=1)


def kernel(x, edge_index, edge_attr, batch, W1, b1, p1, W2, b2, p2, Wl1, bl1, Wl2, bl2, Wl3, bl3):
    src, dst = edge_index[0], edge_index[1]
    n = x.shape[0]
    e = src.shape[0]
    assert n <= _NP and e % (_NW * _K) == 0

    degk = _make_deg_kernel(e)
    msgk = _make_msg_kernel(e)
    nr = _NP // 128

    xp = jnp.pad(x, ((0, _NP - n), (0, 0)))

    def conv(xin, keepf, W, b, p):
        degp, ew = degk(src, dst, edge_attr, keepf.reshape(nr, 128))
        h = _mm(xp if xin is None else xin, W)
        degpt = degp.reshape(_NW, _NP).T
        dinv, g = _prep(degpt, h)
        acc = msgk(g, src, dst, ew)
        return _epi(acc[0], acc[1], h, dinv, b, p)

    onesf = jnp.ones((_NP,), jnp.float32)
    hp1, s1 = conv(None, onesf, W1, b1, p1)

    valid0 = jnp.ones((n,), bool)
    keep1, k1 = _topk_mask(s1[:n, 0], batch, valid0)
    x1 = _readout(hp1[:n], batch, keep1, k1)

    keepf1 = jnp.pad(keep1.astype(jnp.float32), (0, _NP - n))
    hp2, s2 = conv(hp1, keepf1, W2, b2, p2)

    keep2, k2 = _topk_mask(s2[:n, 0], batch, keep1)
    x2 = _readout(hp2[:n], batch, keep2, k2)

    return _head(x1, x2, Wl1, bl1, Wl2, bl2, Wl3, bl3)


# speedup vs baseline: 13.1472x; 1.2579x over previous
import functools

import jax
import jax.numpy as jnp
from jax import lax
from jax.experimental import pallas as pl
from jax.experimental.pallas import tpu as pltpu
from jax.experimental.pallas import tpu_sc as plsc

_G = 128
_RATIO = 0.5
_NC = 2
_NS = 16
_NW = _NC * _NS
_K = 128
_R = 80
_EP = _NW * _R * _K
_NP = 10240
_NR = _NP // 128


def _make_deg_kernel():
    mesh = plsc.VectorSubcoreMesh(core_axis_name="c", subcore_axis_name="s")

    @functools.partial(
        pl.kernel,
        mesh=mesh,
        compiler_params=pltpu.CompilerParams(needs_layout_passes=False),
        out_type=[
            jax.ShapeDtypeStruct((_NW, _NR, 128), jnp.float32),
            jax.ShapeDtypeStruct((_NW, _R, _K), jnp.float32),
        ],
        scratch_types=[
            pltpu.VMEM((_R, _K), jnp.int32),
            pltpu.VMEM((_R, _K), jnp.int32),
            pltpu.VMEM((_R, _K), jnp.float32),
            pltpu.VMEM((_NR, 128), jnp.float32),
            pltpu.VMEM((_NR, 128), jnp.float32),
        ],
    )
    def k(src_hbm, dst_hbm, ew_hbm, keep_hbm, degp_hbm, ew2_hbm,
          s2d, d2d, e2d, keepv, degv):
        cid = lax.axis_index("c")
        sid = lax.axis_index("s")
        wid = cid * _NS + sid
        iota = lax.iota(jnp.int32, 16)
        zeros = jnp.zeros((16,), jnp.float32)

        pltpu.sync_copy(keep_hbm, keepv)
        pltpu.sync_copy(src_hbm.at[wid], s2d)
        pltpu.sync_copy(dst_hbm.at[wid], d2d)
        pltpu.sync_copy(ew_hbm.at[wid], e2d)

        def zb(r, c):
            rsp = jnp.full((16,), r, jnp.int32)
            for f in range(8):
                plsc.store_scatter(degv, [rsp, f * 16 + iota], zeros)
            return c

        lax.fori_loop(0, _NR, zb, 0)

        def rb(r, c):
            rsp = jnp.full((16,), r, jnp.int32)
            for g in range(_K // 16):
                ci = g * 16 + iota
                s16 = plsc.load_gather(s2d, [rsp, ci])
                d16 = plsc.load_gather(d2d, [rsp, ci])
                w16 = plsc.load_gather(e2d, [rsp, ci])
                ks = plsc.load_gather(keepv, [s16 >> 7, s16 & 127])
                kd = plsc.load_gather(keepv, [d16 >> 7, d16 & 127])
                w2 = w16 * ks * kd
                plsc.store_scatter(e2d, [rsp, ci], w2)
                plsc.addupdate_scatter(degv, [d16 >> 7, d16 & 127], w2)
            return c

        lax.fori_loop(0, _R, rb, 0)
        pltpu.sync_copy(degv, degp_hbm.at[wid])
        pltpu.sync_copy(e2d, ew2_hbm.at[wid])

    return k


def _make_msg_kernel():
    nps = _NP // _NS
    zc = 128
    mesh = plsc.VectorSubcoreMesh(core_axis_name="c", subcore_axis_name="s")

    @functools.partial(
        pl.kernel,
        mesh=mesh,
        compiler_params=pltpu.CompilerParams(needs_layout_passes=False),
        out_type=jax.ShapeDtypeStruct((_NC, _NP, 128), jnp.float32),
        scratch_types=[
            pltpu.VMEM((16, _K), jnp.int32),
            pltpu.VMEM((16, _K), jnp.int32),
            pltpu.VMEM((16, _K), jnp.float32),
            pltpu.VMEM((_K, 128), jnp.float32),
            pltpu.VMEM((_K, 128), jnp.float32),
            pltpu.VMEM_SHARED((_NP, 128), jnp.float32),
            pltpu.SemaphoreType.DMA,
            pltpu.SemaphoreType.DMA,
            pltpu.SemaphoreType.DMA,
            pltpu.SemaphoreType.DMA,
        ],
    )
    def k(g_hbm, src_hbm, dst_hbm, ew_hbm, out_hbm,
          s2c, d2c, e2c, rows0, rows1, acc,
          gsem0, gsem1, ssem0, ssem1):
        cid = lax.axis_index("c")
        sid = lax.axis_index("s")
        wid = cid * _NS + sid
        iota = lax.iota(jnp.int32, 16)
        zeros = jnp.zeros((16,), jnp.float32)
        rows = (rows0, rows1)
        gsem = (gsem0, gsem1)
        ssem = (ssem0, ssem1)

        def zb(r, c):
            rsp = jnp.full((16,), r, jnp.int32)
            for f in range(8):
                plsc.store_scatter(rows0, [rsp, f * 16 + iota], zeros)
            return c

        lax.fori_loop(0, zc, zb, 0)
        for t in range(nps // zc):
            pltpu.sync_copy(rows0, acc.at[pl.ds(sid * nps + t * zc, zc)])
        plsc.subcore_barrier()

        def gather(l, b):
            return pltpu.make_async_copy(g_hbm.at[s2c.at[l]], rows[b], gsem[b])

        def scatter(l, b):
            return pltpu.make_async_copy(rows[b], acc.at[d2c.at[l]], ssem[b])

        def scale(l, b):
            def rowb(q, c2):
                rsp = jnp.full((16,), q, jnp.int32)
                coef = plsc.load_gather(e2c, [jnp.full((16,), l, jnp.int32),
                                              rsp])
                for f in range(8):
                    ci = f * 16 + iota
                    v = plsc.load_gather(rows[b], [rsp, ci])
                    plsc.store_scatter(rows[b], [rsp, ci], v * coef)
                return c2

            lax.fori_loop(0, _K, rowb, 0)

        for ch in range(_R // 16):
            pltpu.sync_copy(src_hbm.at[wid, pl.ds(ch * 16, 16)], s2c)
            pltpu.sync_copy(dst_hbm.at[wid, pl.ds(ch * 16, 16)], d2c)
            pltpu.sync_copy(ew_hbm.at[wid, pl.ds(ch * 16, 16)], e2c)
            gather(0, 0).start()

            def rb(j, c):
                for b in range(2):
                    l = 2 * j + b
                    gather(l, b).wait()

                    @pl.when(l >= 1)
                    def _():
                        scatter(l - 1, 1 - b).wait()

                    @pl.when(l + 1 < 16)
                    def _():
                        gather(l + 1, 1 - b).start()

                    scale(l, b)
                    scatter(l, b).start(add=True)
                return c

            lax.fori_loop(0, 8, rb, 0)
            scatter(15, 1).wait()
        plsc.subcore_barrier()
        for t in range(nps // zc):
            sl = pl.ds(sid * nps + t * zc, zc)
            pltpu.sync_copy(acc.at[sl], out_hbm.at[cid, sl])

    return k


def _mm_body(x_ref, w_ref, o_ref):
    o_ref[...] = jnp.dot(x_ref[...], w_ref[...],
                         preferred_element_type=jnp.float32)


def _mm(x, W):
    return pl.pallas_call(
        _mm_body,
        out_shape=jax.ShapeDtypeStruct((x.shape[0], W.shape[1]), jnp.float32),
    )(x, W)


def _prep_body(degpt_ref, h_ref, dinv_ref, g_ref):
    deg = jnp.sum(degpt_ref[...], axis=1, keepdims=True) + 1.0
    dinv = lax.rsqrt(jnp.maximum(deg, 1e-12))
    dinv_ref[...] = dinv
    g_ref[...] = dinv * h_ref[...]


def _prep(degpt, h):
    n = h.shape[0]
    return pl.pallas_call(
        _prep_body,
        out_shape=(jax.ShapeDtypeStruct((n, 1), jnp.float32),
                   jax.ShapeDtypeStruct((n, 128), jnp.float32)),
    )(degpt, h)


def _epi_body(a0_ref, a1_ref, h_ref, dinv_ref, b_ref, p_ref, hp_ref, s_ref):
    dinv = dinv_ref[...]
    out = dinv * (a0_ref[...] + a1_ref[...]) \
        + (dinv * dinv) * h_ref[...] + b_ref[...]
    hf = jnp.maximum(out, 0.0)
    p = p_ref[...]
    pn = p * lax.rsqrt(jnp.sum(p * p))
    s = jnp.tanh(jnp.dot(hf, pn, preferred_element_type=jnp.float32))
    hp_ref[...] = hf * s
    s_ref[...] = s


def _epi(a0, a1, h, dinv, b, p):
    n = h.shape[0]
    return pl.pallas_call(
        _epi_body,
        out_shape=(jax.ShapeDtypeStruct((n, 128), jnp.float32),
                   jax.ShapeDtypeStruct((n, 1), jnp.float32)),
    )(a0, a1, h, dinv, b.reshape(1, -1), p.reshape(-1, 1))


def _head_body(x1_ref, x2_ref, w1_ref, b1_ref, w2_ref, b2_ref, w3_ref, b3_ref, o_ref):
    z = jnp.maximum(x1_ref[...] + x2_ref[...], 0.0)
    z = jnp.maximum(jnp.dot(z, w1_ref[...], preferred_element_type=jnp.float32) + b1_ref[...], 0.0)
    z = jnp.maximum(jnp.dot(z, w2_ref[...], preferred_element_type=jnp.float32) + b2_ref[...], 0.0)
    z = jnp.dot(z, w3_ref[...], preferred_element_type=jnp.float32) + b3_ref[...]
    o_ref[...] = jax.nn.sigmoid(z)


def _head(x1, x2, Wl1, bl1, Wl2, bl2, Wl3, bl3):
    g = x1.shape[0]
    c = Wl3.shape[1]
    return pl.pallas_call(
        _head_body,
        out_shape=jax.ShapeDtypeStruct((g, c), jnp.float32),
    )(x1, x2, Wl1, bl1.reshape(1, -1), Wl2, bl2.reshape(1, -1), Wl3, bl3.reshape(1, -1))


def _topk_mask(score, batch, valid):
    n = score.shape[0]
    ones = jnp.ones_like(batch)
    full = jax.ops.segment_sum(ones, batch, num_segments=_G)
    starts = jnp.cumsum(full) - full
    vcounts = jax.ops.segment_sum(jnp.where(valid, ones, jnp.zeros_like(ones)), batch, num_segments=_G)
    k = jnp.ceil(_RATIO * vcounts).astype(batch.dtype)
    rank = jnp.arange(n, dtype=batch.dtype) - starts[batch]
    keepp = rank < k[batch]
    skey = jnp.where(valid, score, -jnp.inf)
    perm = jnp.lexsort((-skey, batch))
    mask = jnp.zeros((n,), bool).at[perm].set(keepp)
    return mask, k


def _readout(x, batch, valid, k):
    counts = k.astype(jnp.float32)
    mx = jax.ops.segment_max(jnp.where(valid[:, None], x, -jnp.inf), batch, num_segments=_G)
    mx = jnp.where(counts[:, None] > 0, mx, 0.0)
    mean = jax.ops.segment_sum(jnp.where(valid[:, None], x, 0.0), batch, num_segments=_G) / jnp.maximum(counts, 1.0)[:, None]
    return jnp.concatenate([mx, mean], axis=1)


def kernel(x, edge_index, edge_attr, batch, W1, b1, p1, W2, b2, p2, Wl1, bl1, Wl2, bl2, Wl3, bl3):
    src, dst = edge_index[0], edge_index[1]
    n = x.shape[0]
    e = src.shape[0]
    assert n <= _NP and e <= _EP

    degk = _make_deg_kernel()
    msgk = _make_msg_kernel()

    pad_e = _EP - e
    src3 = jnp.pad(src, (0, pad_e)).reshape(_NW, _R, _K)
    dst3 = jnp.pad(dst, (0, pad_e)).reshape(_NW, _R, _K)
    ew3 = jnp.pad(edge_attr, (0, pad_e)).reshape(_NW, _R, _K)
    xp = jnp.pad(x, ((0, _NP - n), (0, 0)))

    def conv(xin, keepf, W, b, p):
        degp, ew2 = degk(src3, dst3, ew3, keepf.reshape(_NR, 128))
        h = _mm(xin, W)
        degpt = degp.reshape(_NW, _NP).T
        dinv, g = _prep(degpt, h)
        acc = msgk(g, src3, dst3, ew2)
        return _epi(acc[0], acc[1], h, dinv, b, p)

    onesf = jnp.ones((_NP,), jnp.float32)
    hp1, s1 = conv(xp, onesf, W1, b1, p1)

    valid0 = jnp.ones((n,), bool)
    keep1, k1 = _topk_mask(s1[:n, 0], batch, valid0)
    x1 = _readout(hp1[:n], batch, keep1, k1)

    keepf1 = jnp.pad(keep1.astype(jnp.float32), (0, _NP - n))
    hp2, s2 = conv(hp1, keepf1, W2, b2, p2)

    keep2, k2 = _topk_mask(s2[:n, 0], batch, keep1)
    x2 = _readout(hp2[:n], batch, keep2, k2)

    return _head(x1, x2, Wl1, bl1, Wl2, bl2, Wl3, bl3)


# speedup vs baseline: 14.7995x; 1.1257x over previous
import functools

import jax
import jax.numpy as jnp
from jax import lax
from jax.experimental import pallas as pl
from jax.experimental.pallas import tpu as pltpu
from jax.experimental.pallas import tpu_sc as plsc

_G = 128
_RATIO = 0.5
_NC = 2
_NS = 16
_NW = _NC * _NS
_K = 128
_R = 80
_EP = _NW * _R * _K
_NP = 10240
_NR = _NP // 128


def _make_deg_kernel():
    mesh = plsc.VectorSubcoreMesh(core_axis_name="c", subcore_axis_name="s")

    @functools.partial(
        pl.kernel,
        mesh=mesh,
        compiler_params=pltpu.CompilerParams(needs_layout_passes=False),
        out_type=[
            jax.ShapeDtypeStruct((_NW, _NR, 128), jnp.float32),
            jax.ShapeDtypeStruct((_NW, _R, _K), jnp.float32),
        ],
        scratch_types=[
            pltpu.VMEM((_R, _K), jnp.int32),
            pltpu.VMEM((_R, _K), jnp.int32),
            pltpu.VMEM((_R, _K), jnp.float32),
            pltpu.VMEM((_NR, 128), jnp.float32),
            pltpu.VMEM((_NR, 128), jnp.float32),
        ],
    )
    def k(src_hbm, dst_hbm, ew_hbm, keep_hbm, degp_hbm, ew2_hbm,
          s2d, d2d, e2d, keepv, degv):
        cid = lax.axis_index("c")
        sid = lax.axis_index("s")
        wid = cid * _NS + sid
        iota = lax.iota(jnp.int32, 16)
        zeros = jnp.zeros((16,), jnp.float32)

        pltpu.sync_copy(keep_hbm, keepv)
        pltpu.sync_copy(src_hbm.at[wid], s2d)
        pltpu.sync_copy(dst_hbm.at[wid], d2d)
        pltpu.sync_copy(ew_hbm.at[wid], e2d)

        def zb(r, c):
            rsp = jnp.full((16,), r, jnp.int32)
            for f in range(8):
                plsc.store_scatter(degv, [rsp, f * 16 + iota], zeros)
            return c

        lax.fori_loop(0, _NR, zb, 0)

        def rb(r, c):
            rsp = jnp.full((16,), r, jnp.int32)
            for g in range(_K // 16):
                ci = g * 16 + iota
                s16 = plsc.load_gather(s2d, [rsp, ci])
                d16 = plsc.load_gather(d2d, [rsp, ci])
                w16 = plsc.load_gather(e2d, [rsp, ci])
                ks = plsc.load_gather(keepv, [s16 >> 7, s16 & 127])
                kd = plsc.load_gather(keepv, [d16 >> 7, d16 & 127])
                w2 = w16 * ks * kd
                plsc.store_scatter(e2d, [rsp, ci], w2)
                plsc.addupdate_scatter(degv, [d16 >> 7, d16 & 127], w2)
            return c

        lax.fori_loop(0, _R, rb, 0)
        pltpu.sync_copy(degv, degp_hbm.at[wid])
        pltpu.sync_copy(e2d, ew2_hbm.at[wid])

    return k


def _make_msg_kernel():
    nps = _NP // _NS
    zc = 128
    mesh = plsc.VectorSubcoreMesh(core_axis_name="c", subcore_axis_name="s")

    @functools.partial(
        pl.kernel,
        mesh=mesh,
        compiler_params=pltpu.CompilerParams(needs_layout_passes=False),
        out_type=jax.ShapeDtypeStruct((_NC, _NP, 128), jnp.float32),
        scratch_types=[
            pltpu.VMEM((16, _K), jnp.int32),
            pltpu.VMEM((16, _K), jnp.int32),
            pltpu.VMEM((16, _K), jnp.float32),
            pltpu.VMEM((_K, 128), jnp.float32),
            pltpu.VMEM((_K, 128), jnp.float32),
            pltpu.VMEM_SHARED((_NP, 128), jnp.float32),
            pltpu.SemaphoreType.DMA,
            pltpu.SemaphoreType.DMA,
            pltpu.SemaphoreType.DMA,
            pltpu.SemaphoreType.DMA,
        ],
    )
    def k(g_hbm, src_hbm, dst_hbm, ew_hbm, out_hbm,
          s2c, d2c, e2c, rows0, rows1, acc,
          gsem0, gsem1, ssem0, ssem1):
        cid = lax.axis_index("c")
        sid = lax.axis_index("s")
        wid = cid * _NS + sid
        iota = lax.iota(jnp.int32, 16)
        zeros = jnp.zeros((16,), jnp.float32)
        rows = (rows0, rows1)
        gsem = (gsem0, gsem1)
        ssem = (ssem0, ssem1)

        @functools.partial(plsc.parallel_loop, 0, zc, unroll=4)
        def zb(r):
            rsp = jnp.full((16,), r, jnp.int32)
            for f in range(8):
                plsc.store_scatter(rows0, [rsp, f * 16 + iota], zeros)
        for t in range(nps // zc):
            pltpu.sync_copy(rows0, acc.at[pl.ds(sid * nps + t * zc, zc)])
        plsc.subcore_barrier()

        def gather(l, b):
            return pltpu.make_async_copy(g_hbm.at[s2c.at[l]], rows[b], gsem[b])

        def scatter(l, b):
            return pltpu.make_async_copy(rows[b], acc.at[d2c.at[l]], ssem[b])

        def scale(l, b):
            @functools.partial(plsc.parallel_loop, 0, _K, unroll=4)
            def rowb(q):
                rsp = jnp.full((16,), q, jnp.int32)
                coef = plsc.load_gather(e2c, [jnp.full((16,), l, jnp.int32),
                                              rsp])
                for f in range(8):
                    ci = f * 16 + iota
                    v = plsc.load_gather(rows[b], [rsp, ci])
                    plsc.store_scatter(rows[b], [rsp, ci], v * coef)

        for ch in range(_R // 16):
            pltpu.sync_copy(src_hbm.at[wid, pl.ds(ch * 16, 16)], s2c)
            pltpu.sync_copy(dst_hbm.at[wid, pl.ds(ch * 16, 16)], d2c)
            pltpu.sync_copy(ew_hbm.at[wid, pl.ds(ch * 16, 16)], e2c)
            gather(0, 0).start()

            def rb(j, c):
                for b in range(2):
                    l = 2 * j + b
                    gather(l, b).wait()

                    @pl.when(l >= 1)
                    def _():
                        scatter(l - 1, 1 - b).wait()

                    @pl.when(l + 1 < 16)
                    def _():
                        gather(l + 1, 1 - b).start()

                    scale(l, b)
                    scatter(l, b).start(add=True)
                return c

            lax.fori_loop(0, 8, rb, 0)
            scatter(15, 1).wait()
        plsc.subcore_barrier()
        for t in range(nps // zc):
            sl = pl.ds(sid * nps + t * zc, zc)
            pltpu.sync_copy(acc.at[sl], out_hbm.at[cid, sl])

    return k


def _mm_body(x_ref, w_ref, o_ref):
    o_ref[...] = jnp.dot(x_ref[...], w_ref[...],
                         preferred_element_type=jnp.float32)


def _mm(x, W):
    return pl.pallas_call(
        _mm_body,
        out_shape=jax.ShapeDtypeStruct((x.shape[0], W.shape[1]), jnp.float32),
    )(x, W)


def _prep_body(degpt_ref, h_ref, dinv_ref, g_ref):
    deg = jnp.sum(degpt_ref[...], axis=1, keepdims=True) + 1.0
    dinv = lax.rsqrt(jnp.maximum(deg, 1e-12))
    dinv_ref[...] = dinv
    g_ref[...] = dinv * h_ref[...]


def _prep(degpt, h):
    n = h.shape[0]
    return pl.pallas_call(
        _prep_body,
        out_shape=(jax.ShapeDtypeStruct((n, 1), jnp.float32),
                   jax.ShapeDtypeStruct((n, 128), jnp.float32)),
    )(degpt, h)


def _epi_body(a0_ref, a1_ref, h_ref, dinv_ref, b_ref, p_ref, hp_ref, s_ref):
    dinv = dinv_ref[...]
    out = dinv * (a0_ref[...] + a1_ref[...]) \
        + (dinv * dinv) * h_ref[...] + b_ref[...]
    hf = jnp.maximum(out, 0.0)
    p = p_ref[...]
    pn = p * lax.rsqrt(jnp.sum(p * p))
    s = jnp.tanh(jnp.dot(hf, pn, preferred_element_type=jnp.float32))
    hp_ref[...] = hf * s
    s_ref[...] = s


def _epi(a0, a1, h, dinv, b, p):
    n = h.shape[0]
    return pl.pallas_call(
        _epi_body,
        out_shape=(jax.ShapeDtypeStruct((n, 128), jnp.float32),
                   jax.ShapeDtypeStruct((n, 1), jnp.float32)),
    )(a0, a1, h, dinv, b.reshape(1, -1), p.reshape(-1, 1))


def _head_body(x1_ref, x2_ref, w1_ref, b1_ref, w2_ref, b2_ref, w3_ref, b3_ref, o_ref):
    z = jnp.maximum(x1_ref[...] + x2_ref[...], 0.0)
    z = jnp.maximum(jnp.dot(z, w1_ref[...], preferred_element_type=jnp.float32) + b1_ref[...], 0.0)
    z = jnp.maximum(jnp.dot(z, w2_ref[...], preferred_element_type=jnp.float32) + b2_ref[...], 0.0)
    z = jnp.dot(z, w3_ref[...], preferred_element_type=jnp.float32) + b3_ref[...]
    o_ref[...] = jax.nn.sigmoid(z)


def _head(x1, x2, Wl1, bl1, Wl2, bl2, Wl3, bl3):
    g = x1.shape[0]
    c = Wl3.shape[1]
    return pl.pallas_call(
        _head_body,
        out_shape=jax.ShapeDtypeStruct((g, c), jnp.float32),
    )(x1, x2, Wl1, bl1.reshape(1, -1), Wl2, bl2.reshape(1, -1), Wl3, bl3.reshape(1, -1))


def _topk_mask(score, batch, valid):
    n = score.shape[0]
    ones = jnp.ones_like(batch)
    full = jax.ops.segment_sum(ones, batch, num_segments=_G)
    starts = jnp.cumsum(full) - full
    vcounts = jax.ops.segment_sum(jnp.where(valid, ones, jnp.zeros_like(ones)), batch, num_segments=_G)
    k = jnp.ceil(_RATIO * vcounts).astype(batch.dtype)
    rank = jnp.arange(n, dtype=batch.dtype) - starts[batch]
    keepp = rank < k[batch]
    skey = jnp.where(valid, score, -jnp.inf)
    perm = jnp.lexsort((-skey, batch))
    mask = jnp.zeros((n,), bool).at[perm].set(keepp)
    return mask, k


def _readout(x, batch, valid, k):
    counts = k.astype(jnp.float32)
    mx = jax.ops.segment_max(jnp.where(valid[:, None], x, -jnp.inf), batch, num_segments=_G)
    mx = jnp.where(counts[:, None] > 0, mx, 0.0)
    mean = jax.ops.segment_sum(jnp.where(valid[:, None], x, 0.0), batch, num_segments=_G) / jnp.maximum(counts, 1.0)[:, None]
    return jnp.concatenate([mx, mean], axis=1)


def kernel(x, edge_index, edge_attr, batch, W1, b1, p1, W2, b2, p2, Wl1, bl1, Wl2, bl2, Wl3, bl3):
    src, dst = edge_index[0], edge_index[1]
    n = x.shape[0]
    e = src.shape[0]
    assert n <= _NP and e <= _EP

    degk = _make_deg_kernel()
    msgk = _make_msg_kernel()

    pad_e = _EP - e
    src3 = jnp.pad(src, (0, pad_e)).reshape(_NW, _R, _K)
    dst3 = jnp.pad(dst, (0, pad_e)).reshape(_NW, _R, _K)
    ew3 = jnp.pad(edge_attr, (0, pad_e)).reshape(_NW, _R, _K)
    xp = jnp.pad(x, ((0, _NP - n), (0, 0)))

    def conv(xin, keepf, W, b, p):
        degp, ew2 = degk(src3, dst3, ew3, keepf.reshape(_NR, 128))
        h = _mm(xin, W)
        degpt = degp.reshape(_NW, _NP).T
        dinv, g = _prep(degpt, h)
        acc = msgk(g, src3, dst3, ew2)
        return _epi(acc[0], acc[1], h, dinv, b, p)

    onesf = jnp.ones((_NP,), jnp.float32)
    hp1, s1 = conv(xp, onesf, W1, b1, p1)

    valid0 = jnp.ones((n,), bool)
    keep1, k1 = _topk_mask(s1[:n, 0], batch, valid0)
    x1 = _readout(hp1[:n], batch, keep1, k1)

    keepf1 = jnp.pad(keep1.astype(jnp.float32), (0, _NP - n))
    hp2, s2 = conv(hp1, keepf1, W2, b2, p2)

    keep2, k2 = _topk_mask(s2[:n, 0], batch, keep1)
    x2 = _readout(hp2[:n], batch, keep2, k2)

    return _head(x1, x2, Wl1, bl1, Wl2, bl2, Wl3, bl3)
